# baseline (device time: 474607 ns/iter reference)
import jax
import jax.numpy as jnp
from jax import lax
from jax.experimental import pallas as pl
from jax.experimental.pallas import tpu as pltpu

T = 1024
D = 2048
VL = 16384
CH = 1024
NCH = VL // CH
K_EAGER = 4


def _body_a(x_ref, w_hbm, out_hbm, e_recv_hbm, e_loc_hbm, inv_ref,
            x_bf, w_vmem, proc, out_stage, s_send, s_recv,
            send_sems, recv_sems, s_send_sem, s_recv_sem,
            w_sems, st_sems, rd_sems, out_sems):
    my_x = lax.axis_index("x")
    my_y = lax.axis_index("y")
    peer = (my_x, 1 - my_y)

    barrier = pltpu.get_barrier_semaphore()
    pl.semaphore_signal(
        barrier, inc=1, device_id=peer,
        device_id_type=pl.DeviceIdType.MESH,
    )
    pl.semaphore_wait(barrier, 1)

    x_bf[...] = x_ref[...].astype(jnp.bfloat16)

    def w_load(j, slot):
        return pltpu.make_async_copy(
            w_hbm.at[:, pl.ds(j * CH, CH)], w_vmem.at[slot],
            w_sems.at[slot])

    w_load(0, 0).start()
    s_l = jnp.zeros((T, 1), jnp.float32)
    descs = []
    stash = []
    stash_waited = []
    for j in range(NCH):
        slot = j % 2
        if j + 1 < NCH:
            w_load(j + 1, 1 - slot).start()
        w_load(j, slot).wait()
        logits = jnp.dot(
            x_bf[...], w_vmem[slot].astype(jnp.bfloat16),
            preferred_element_type=jnp.float32)
        e = jnp.exp(logits)
        s_l = s_l + jnp.sum(e, axis=1, keepdims=True)
        if j >= 2 and not stash_waited[j - 2]:
            stash[j - 2].wait()
            stash_waited[j - 2] = True
        proc[slot, :, :] = e.astype(jnp.bfloat16)
        st = pltpu.make_async_copy(
            proc.at[slot], e_loc_hbm.at[j], st_sems.at[slot])
        st.start()
        stash.append(st)
        stash_waited.append(False)
        rdma = pltpu.make_async_remote_copy(
            src_ref=e_loc_hbm.at[j],
            dst_ref=e_recv_hbm.at[j],
            send_sem=send_sems.at[j],
            recv_sem=recv_sems.at[j],
            device_id=peer,
            device_id_type=pl.DeviceIdType.MESH,
        )
        descs.append(rdma)
        if j < K_EAGER:
            st.wait()
            stash_waited[j] = True
            rdma.start()
    for j in range(NCH):
        if not stash_waited[j]:
            stash[j].wait()
            stash_waited[j] = True

    s_send[...] = jnp.broadcast_to(s_l, (T, 8))
    s_rdma = pltpu.make_async_remote_copy(
        src_ref=s_send, dst_ref=s_recv,
        send_sem=s_send_sem, recv_sem=s_recv_sem,
        device_id=peer, device_id_type=pl.DeviceIdType.MESH,
    )
    s_rdma.start()
    for j in range(K_EAGER, NCH):
        descs[j].start()

    s_rdma.wait_recv()
    inv = 1.0 / (s_l + s_recv[:, 0:1])
    inv_ref[...] = jnp.broadcast_to(inv, (T, 128))

    ocs = []
    for half in range(2):
        base = (my_y if half == 0 else 1 - my_y) * VL
        src = e_loc_hbm if half == 0 else e_recv_hbm
        for j in range(NCH):
            slot = j % 2
            if half == 1:
                descs[j].wait_recv()
                if j == NCH - 1:
                    break
            rc = pltpu.make_async_copy(
                src.at[j], proc.at[slot], rd_sems.at[slot])
            rc.start()
            rc.wait()
            if len(ocs) >= 2:
                ocs[len(ocs) - 2].wait()
            out_stage[slot, :, :] = (
                proc[slot].astype(jnp.float32) * inv)
            oc = pltpu.make_async_copy(
                out_stage.at[slot],
                out_hbm.at[:, pl.ds(base + j * CH, CH)],
                out_sems.at[slot])
            oc.start()
            ocs.append(oc)
    ocs[-2].wait()
    ocs[-1].wait()

    for j in range(NCH):
        descs[j].wait_send()
    s_rdma.wait_send()


def _body_b(out_in, e_recv_hbm, inv_ref, out_hbm,
            proc, out_stage, rd_sem, out_sem):
    del out_in
    my_y = lax.axis_index("y")
    j = NCH - 1
    rc = pltpu.make_async_copy(e_recv_hbm.at[j], proc, rd_sem)
    rc.start()
    rc.wait()
    out_stage[...] = proc[...].astype(jnp.float32) * inv_ref[:, 0:1]
    oc = pltpu.make_async_copy(
        out_stage,
        out_hbm.at[:, pl.ds((1 - my_y) * VL + j * CH, CH)],
        out_sem)
    oc.start()
    oc.wait()


def kernel(x, W):
    out_a, e_recv, _, inv = pl.pallas_call(
        _body_a,
        out_shape=(
            jax.ShapeDtypeStruct((T, 2 * VL), jnp.float32),
            jax.ShapeDtypeStruct((NCH, T, CH), jnp.bfloat16),
            jax.ShapeDtypeStruct((NCH, T, CH), jnp.bfloat16),
            jax.ShapeDtypeStruct((T, 128), jnp.float32),
        ),
        in_specs=[
            pl.BlockSpec(memory_space=pltpu.VMEM),
            pl.BlockSpec(memory_space=pl.ANY),
        ],
        out_specs=[
            pl.BlockSpec(memory_space=pl.ANY),
            pl.BlockSpec(memory_space=pl.ANY),
            pl.BlockSpec(memory_space=pl.ANY),
            pl.BlockSpec(memory_space=pltpu.VMEM),
        ],
        scratch_shapes=[
            pltpu.VMEM((T, D), jnp.bfloat16),
            pltpu.VMEM((2, D, CH), jnp.float32),
            pltpu.VMEM((2, T, CH), jnp.bfloat16),
            pltpu.VMEM((2, T, CH), jnp.float32),
            pltpu.VMEM((T, 8), jnp.float32),
            pltpu.VMEM((T, 8), jnp.float32),
            pltpu.SemaphoreType.DMA((NCH,)),
            pltpu.SemaphoreType.DMA((NCH,)),
            pltpu.SemaphoreType.DMA,
            pltpu.SemaphoreType.DMA,
            pltpu.SemaphoreType.DMA((2,)),
            pltpu.SemaphoreType.DMA((2,)),
            pltpu.SemaphoreType.DMA((2,)),
            pltpu.SemaphoreType.DMA((2,)),
        ],
        compiler_params=pltpu.CompilerParams(
            collective_id=0,
            vmem_limit_bytes=100 * 1024 * 1024,
        ),
    )(x, W)

    out = pl.pallas_call(
        _body_b,
        out_shape=jax.ShapeDtypeStruct((T, 2 * VL), jnp.float32),
        in_specs=[
            pl.BlockSpec(memory_space=pl.ANY),
            pl.BlockSpec(memory_space=pl.ANY),
            pl.BlockSpec(memory_space=pltpu.VMEM),
        ],
        out_specs=pl.BlockSpec(memory_space=pl.ANY),
        scratch_shapes=[
            pltpu.VMEM((T, CH), jnp.bfloat16),
            pltpu.VMEM((T, CH), jnp.float32),
            pltpu.SemaphoreType.DMA,
            pltpu.SemaphoreType.DMA,
        ],
        input_output_aliases={0: 0},
        compiler_params=pltpu.CompilerParams(
            vmem_limit_bytes=100 * 1024 * 1024,
        ),
    )(out_a, e_recv, inv)
    return out


# device time: 449276 ns/iter; 1.0564x vs baseline; 1.0564x over previous
import jax
import jax.numpy as jnp
from jax import lax
from jax.experimental import pallas as pl
from jax.experimental.pallas import tpu as pltpu

T = 1024
D = 2048
VL = 16384
CH = 1024
NCH = VL // CH
K_EAGER = 4


def _body_a(x_ref, w_hbm, e_recv_hbm, e_loc_hbm, inv_ref,
            x_bf, w_vmem, proc, s_send, s_recv,
            send_sems, recv_sems, s_send_sem, s_recv_sem,
            w_sems, st_sems):
    my_x = lax.axis_index("x")
    my_y = lax.axis_index("y")
    peer = (my_x, 1 - my_y)

    barrier = pltpu.get_barrier_semaphore()
    pl.semaphore_signal(
        barrier, inc=1, device_id=peer,
        device_id_type=pl.DeviceIdType.MESH,
    )
    pl.semaphore_wait(barrier, 1)

    x_bf[...] = x_ref[...].astype(jnp.bfloat16)

    def w_load(j, slot):
        return pltpu.make_async_copy(
            w_hbm.at[:, pl.ds(j * CH, CH)], w_vmem.at[slot],
            w_sems.at[slot])

    w_load(0, 0).start()
    s_l = jnp.zeros((T, 1), jnp.float32)
    descs = []
    stash = []
    stash_waited = []
    for j in range(NCH):
        slot = j % 2
        if j + 1 < NCH:
            w_load(j + 1, 1 - slot).start()
        w_load(j, slot).wait()
        logits = jnp.dot(
            x_bf[...], w_vmem[slot].astype(jnp.bfloat16),
            preferred_element_type=jnp.float32)
        e = jnp.exp(logits)
        s_l = s_l + jnp.sum(e, axis=1, keepdims=True)
        if j >= 2 and not stash_waited[j - 2]:
            stash[j - 2].wait()
            stash_waited[j - 2] = True
        proc[slot, :, :] = e.astype(jnp.bfloat16)
        st = pltpu.make_async_copy(
            proc.at[slot], e_loc_hbm.at[j], st_sems.at[slot])
        st.start()
        stash.append(st)
        stash_waited.append(False)
        rdma = pltpu.make_async_remote_copy(
            src_ref=e_loc_hbm.at[j],
            dst_ref=e_recv_hbm.at[j],
            send_sem=send_sems.at[j],
            recv_sem=recv_sems.at[j],
            device_id=peer,
            device_id_type=pl.DeviceIdType.MESH,
        )
        descs.append(rdma)
        if j < K_EAGER:
            st.wait()
            stash_waited[j] = True
            rdma.start()
    for j in range(NCH):
        if not stash_waited[j]:
            stash[j].wait()
            stash_waited[j] = True

    s_send[...] = jnp.broadcast_to(s_l, (T, 8))
    s_rdma = pltpu.make_async_remote_copy(
        src_ref=s_send, dst_ref=s_recv,
        send_sem=s_send_sem, recv_sem=s_recv_sem,
        device_id=peer, device_id_type=pl.DeviceIdType.MESH,
    )
    s_rdma.start()
    for j in range(K_EAGER, NCH):
        descs[j].start()

    s_rdma.wait_recv()
    inv = 1.0 / (s_l + s_recv[:, 0:1])
    inv_ref[...] = jnp.broadcast_to(inv, (T, 128))

    for j in range(NCH):
        descs[j].wait_recv()
    for j in range(NCH):
        descs[j].wait_send()
    s_rdma.wait_send()


_NSLOT = 4


def _body_b(e_loc_hbm, e_recv_hbm, inv_ref, out_hbm,
            proc, stage, rd_sems, out_sems):
    my_y = lax.axis_index("y")
    inv = inv_ref[:, 0:1]

    work = [(e_loc_hbm, j, 0) for j in range(NCH)] + [
        (e_recv_hbm, j, 1) for j in range(NCH)]

    reads = []
    for k, (src, j, half) in enumerate(work):
        slot = k % _NSLOT
        rc = pltpu.make_async_copy(src.at[j], proc.at[slot],
                                   rd_sems.at[slot])
        if k < _NSLOT:
            rc.start()
        reads.append(rc)

    ocs = []
    for k, (src, j, half) in enumerate(work):
        slot = k % _NSLOT
        reads[k].wait()
        if k >= _NSLOT:
            ocs[k - _NSLOT].wait()
        base = (my_y if half == 0 else 1 - my_y) * VL
        stage[slot, :, :] = proc[slot].astype(jnp.float32) * inv
        oc = pltpu.make_async_copy(
            stage.at[slot],
            out_hbm.at[:, pl.ds(base + j * CH, CH)],
            out_sems.at[slot])
        oc.start()
        ocs.append(oc)
        if k + _NSLOT < len(work):
            reads[k + _NSLOT].start()
    for k in range(len(work) - _NSLOT, len(work)):
        ocs[k].wait()


def kernel(x, W):
    e_recv, e_loc, inv = pl.pallas_call(
        _body_a,
        out_shape=(
            jax.ShapeDtypeStruct((NCH, T, CH), jnp.bfloat16),
            jax.ShapeDtypeStruct((NCH, T, CH), jnp.bfloat16),
            jax.ShapeDtypeStruct((T, 128), jnp.float32),
        ),
        in_specs=[
            pl.BlockSpec(memory_space=pltpu.VMEM),
            pl.BlockSpec(memory_space=pl.ANY),
        ],
        out_specs=[
            pl.BlockSpec(memory_space=pl.ANY),
            pl.BlockSpec(memory_space=pl.ANY),
            pl.BlockSpec(memory_space=pltpu.VMEM),
        ],
        scratch_shapes=[
            pltpu.VMEM((T, D), jnp.bfloat16),
            pltpu.VMEM((2, D, CH), jnp.float32),
            pltpu.VMEM((2, T, CH), jnp.bfloat16),
            pltpu.VMEM((T, 8), jnp.float32),
            pltpu.VMEM((T, 8), jnp.float32),
            pltpu.SemaphoreType.DMA((NCH,)),
            pltpu.SemaphoreType.DMA((NCH,)),
            pltpu.SemaphoreType.DMA,
            pltpu.SemaphoreType.DMA,
            pltpu.SemaphoreType.DMA((2,)),
            pltpu.SemaphoreType.DMA((2,)),
        ],
        compiler_params=pltpu.CompilerParams(
            collective_id=0,
            vmem_limit_bytes=100 * 1024 * 1024,
        ),
    )(x, W)

    out = pl.pallas_call(
        _body_b,
        out_shape=jax.ShapeDtypeStruct((T, 2 * VL), jnp.float32),
        in_specs=[
            pl.BlockSpec(memory_space=pl.ANY),
            pl.BlockSpec(memory_space=pl.ANY),
            pl.BlockSpec(memory_space=pltpu.VMEM),
        ],
        out_specs=pl.BlockSpec(memory_space=pl.ANY),
        scratch_shapes=[
            pltpu.VMEM((_NSLOT, T, CH), jnp.bfloat16),
            pltpu.VMEM((_NSLOT, T, CH), jnp.float32),
            pltpu.SemaphoreType.DMA((_NSLOT,)),
            pltpu.SemaphoreType.DMA((_NSLOT,)),
        ],
        compiler_params=pltpu.CompilerParams(
            vmem_limit_bytes=100 * 1024 * 1024,
        ),
    )(e_loc, e_recv, inv)
    return out
